# Initial kernel scaffold; baseline (speedup 1.0000x reference)
import jax
import jax.numpy as jnp
from jax.experimental import pallas as pl


def kernel(x, Wg, bg, W1, b1, W2, b2):
    raise NotImplementedError("write your pallas kernel here")


# speedup vs baseline: 3.8172x; 3.8172x over previous
import jax
import jax.numpy as jnp
from jax.experimental import pallas as pl
from jax.experimental.pallas import tpu as pltpu


def _gate_body(x_ref, wg_ref, bg_ref, idx_ref, psum_ref):
    i = pl.program_id(0)
    x = x_ref[...]
    logits = jnp.dot(x, wg_ref[...], preferred_element_type=jnp.float32)
    logits = logits + bg_ref[...]
    m = jnp.max(logits, axis=-1, keepdims=True)
    p = jnp.exp(logits - m)
    probs = p / jnp.sum(p, axis=-1, keepdims=True)
    lane = jax.lax.broadcasted_iota(jnp.int32, logits.shape, 1)
    nexp = logits.shape[-1]
    idx = jnp.min(jnp.where(logits == m, lane, nexp), axis=-1)
    idx_ref[0, 0, :] = idx

    @pl.when(i == 0)
    def _():
        psum_ref[...] = jnp.zeros_like(psum_ref)

    psum_ref[...] += jnp.sum(probs, axis=0, keepdims=True)


def _route_body(idx_ref, tm_ref, cb0_ref, ctot_ref, carry_ref, ke_ref, *,
                nexp, tb, cap):
    i = pl.program_id(0)

    @pl.when(i == 0)
    def _():
        carry_ref[...] = jnp.zeros_like(carry_ref)

    idxv = idx_ref[0, :, :]
    eio = jax.lax.broadcasted_iota(jnp.int32, (nexp, tb), 0)
    onehot_t = (idxv == eio).astype(jnp.float32)
    r = jax.lax.broadcasted_iota(jnp.int32, (tb, tb), 0)
    c = jax.lax.broadcasted_iota(jnp.int32, (tb, tb), 1)
    upper = (r < c).astype(jnp.float32)
    rank = jnp.dot(onehot_t, upper, preferred_element_type=jnp.float32)
    rank = rank + carry_ref[...]
    kept = (onehot_t > 0.5) & (rank < float(cap))
    ke = jnp.max(jnp.where(kept, eio, -1), axis=0, keepdims=True)
    carry_ref[...] += jnp.sum(onehot_t, axis=1, keepdims=True)

    @pl.when(i < 4)
    def _():
        tm_ref[0, :, :] = ke

    for k in range(4):
        @pl.when(i == k)
        def _(k=k):
            ke_ref[k:k + 1, :] = ke

        @pl.when(i == k + 4)
        def _(k=k):
            tm_ref[0, :, :] = jnp.maximum(ke_ref[k:k + 1, :], ke)

    @pl.when(i == 3)
    def _():
        cb0_ref[...] = carry_ref[...]

    @pl.when(i == 7)
    def _():
        ctot_ref[...] = carry_ref[...]


def _ffn_body(x_ref, w1_ref, b1_ref, w2_ref, b2_ref, tm_ref, out_ref):
    e = pl.program_id(1)

    @pl.when(e == 0)
    def _():
        out_ref[...] = jnp.zeros_like(out_ref)

    x = x_ref[...]
    h = jnp.dot(x, w1_ref[0], preferred_element_type=jnp.float32)
    h = jnp.maximum(h + b1_ref[0], 0.0)
    y = jnp.dot(h, w2_ref[0], preferred_element_type=jnp.float32)
    y = y + b2_ref[0]
    sel = tm_ref[...] == e
    out_ref[...] += jnp.where(sel, y, 0.0)


def kernel(x, Wg, bg, W1, b1, W2, b2):
    b, s, d = x.shape
    nexp = Wg.shape[1]
    total = b * s
    cap = int(total / nexp * 1.25)
    nt = 8
    tb = total // nt
    xf = x.reshape(total, d)

    idx8, psum = pl.pallas_call(
        _gate_body,
        grid=(nt,),
        in_specs=[
            pl.BlockSpec((tb, d), lambda i: (i, 0)),
            pl.BlockSpec((d, nexp), lambda i: (0, 0)),
            pl.BlockSpec((1, nexp), lambda i: (0, 0)),
        ],
        out_specs=[
            pl.BlockSpec((1, 1, tb), lambda i: (i, 0, 0)),
            pl.BlockSpec((1, nexp), lambda i: (0, 0)),
        ],
        out_shape=[
            jax.ShapeDtypeStruct((nt, 1, tb), jnp.int32),
            jax.ShapeDtypeStruct((1, nexp), jnp.float32),
        ],
    )(xf, Wg, bg.reshape(1, nexp))

    import functools
    route = functools.partial(_route_body, nexp=nexp, tb=tb, cap=cap)
    tm8, cb0, ctot = pl.pallas_call(
        route,
        grid=(nt,),
        in_specs=[pl.BlockSpec((1, 1, tb), lambda i: (i, 0, 0))],
        out_specs=[
            pl.BlockSpec((1, 1, tb), lambda i: (i, 0, 0)),
            pl.BlockSpec((nexp, 1), lambda i: (0, 0)),
            pl.BlockSpec((nexp, 1), lambda i: (0, 0)),
        ],
        out_shape=[
            jax.ShapeDtypeStruct((nt, 1, tb), jnp.int32),
            jax.ShapeDtypeStruct((nexp, 1), jnp.float32),
            jax.ShapeDtypeStruct((nexp, 1), jnp.float32),
        ],
        scratch_shapes=[
            pltpu.VMEM((nexp, 1), jnp.float32),
            pltpu.VMEM((4, tb), jnp.int32),
        ],
    )(idx8)

    cb0f = cb0.reshape(nexp)
    ctotf = ctot.reshape(nexp)
    earr = jnp.arange(nexp)
    a0 = jnp.max(jnp.where(cb0f > 0.5, earr, -1))
    a1 = jnp.max(jnp.where((ctotf - cb0f > 0.5) & (cb0f < cap), earr, -1))
    tm = tm8.reshape(nt, tb)[nt // 2:].reshape(s)
    tm = tm.at[0].set(jnp.maximum(tm[0], a0))
    tm = tm.at[1].set(jnp.maximum(tm[1], a1))
    tmc = tm.reshape(s, 1)

    tc_r = s // 8
    y0 = pl.pallas_call(
        _ffn_body,
        grid=(s // tc_r, nexp),
        in_specs=[
            pl.BlockSpec((tc_r, d), lambda i, e: (i, 0)),
            pl.BlockSpec((1, d, d), lambda i, e: (e, 0, 0)),
            pl.BlockSpec((1, 1, d), lambda i, e: (e, 0, 0)),
            pl.BlockSpec((1, d, d), lambda i, e: (e, 0, 0)),
            pl.BlockSpec((1, 1, d), lambda i, e: (e, 0, 0)),
            pl.BlockSpec((tc_r, 1), lambda i, e: (i, 0)),
        ],
        out_specs=pl.BlockSpec((tc_r, d), lambda i, e: (i, 0)),
        out_shape=jax.ShapeDtypeStruct((s, d), jnp.float32),
    )(x[0], W1, b1.reshape(nexp, 1, d), W2, b2.reshape(nexp, 1, d), tmc)

    out = jnp.concatenate([y0[None], jnp.zeros_like(y0)[None]], axis=0)
    probs_mean = psum.reshape(nexp) / total
    density = ctotf / total
    expert_loss = jnp.mean(probs_mean * density * (nexp ** 2))
    return out, expert_loss


# speedup vs baseline: 5.9556x; 1.5602x over previous
import functools

import jax
import jax.numpy as jnp
from jax.experimental import pallas as pl
from jax.experimental.pallas import tpu as pltpu


def _gate_body(x_ref, wg_ref, bg_ref, idx_ref, psum_ref):
    i = pl.program_id(0)
    x = x_ref[...]
    logits = jnp.dot(x, wg_ref[...], preferred_element_type=jnp.float32)
    logits = logits + bg_ref[...]
    m = jnp.max(logits, axis=-1, keepdims=True)
    p = jnp.exp(logits - m)
    probs = p / jnp.sum(p, axis=-1, keepdims=True)
    lane = jax.lax.broadcasted_iota(jnp.int32, logits.shape, 1)
    nexp = logits.shape[-1]
    idx = jnp.min(jnp.where(logits == m, lane, nexp), axis=-1)
    idx_ref[0, 0, :] = idx

    @pl.when(i == 0)
    def _():
        psum_ref[...] = jnp.zeros_like(psum_ref)

    psum_ref[...] += jnp.sum(probs, axis=0, keepdims=True)


def _route_body(idx_ref, tm_ref, cb0_ref, ctot_ref, carry_ref, ke_ref, *,
                nexp, tb, cap):
    i = pl.program_id(0)

    @pl.when(i == 0)
    def _():
        carry_ref[...] = jnp.zeros_like(carry_ref)

    idxv = idx_ref[0, :, :]
    eio = jax.lax.broadcasted_iota(jnp.int32, (nexp, tb), 0)
    onehot_t = (idxv == eio).astype(jnp.float32)
    r = jax.lax.broadcasted_iota(jnp.int32, (tb, tb), 0)
    c = jax.lax.broadcasted_iota(jnp.int32, (tb, tb), 1)
    upper = (r < c).astype(jnp.float32)
    rank = jnp.dot(onehot_t, upper, preferred_element_type=jnp.float32)
    rank = rank + carry_ref[...]
    kept = (onehot_t > 0.5) & (rank < float(cap))
    ke = jnp.max(jnp.where(kept, eio, -1), axis=0, keepdims=True)
    carry_ref[...] += jnp.sum(onehot_t, axis=1, keepdims=True)

    @pl.when(i < 4)
    def _():
        tm_ref[0, :, :] = ke

    for k in range(4):
        @pl.when(i == k)
        def _(k=k):
            ke_ref[k:k + 1, :] = ke

        @pl.when(i == k + 4)
        def _(k=k):
            tm_ref[0, :, :] = jnp.maximum(ke_ref[k:k + 1, :], ke)

    @pl.when(i == 3)
    def _():
        cb0_ref[...] = carry_ref[...]

    @pl.when(i == 7)
    def _():
        ctot_ref[...] = carry_ref[...]


def _group_body(tm_ref, slot_ref, cnt2_ref, off_ref, carry_ref, rank_ref, *,
                nexp, tb2, blk, nslot):
    i = pl.program_id(0)

    @pl.when(i == 0)
    def _():
        carry_ref[...] = jnp.zeros_like(carry_ref)

    tmv = tm_ref[0, :, :]
    eio = jax.lax.broadcasted_iota(jnp.int32, (nexp, tb2), 0)
    onehot_t = (tmv == eio).astype(jnp.float32)

    @pl.when(i < 4)
    def _():
        r = jax.lax.broadcasted_iota(jnp.int32, (tb2, tb2), 0)
        c = jax.lax.broadcasted_iota(jnp.int32, (tb2, tb2), 1)
        upper = (r < c).astype(jnp.float32)
        rank = jnp.dot(onehot_t, upper, preferred_element_type=jnp.float32)
        rank = rank + carry_ref[...]
        rank_tok = jnp.sum(onehot_t * rank, axis=0, keepdims=True)
        for k in range(4):
            @pl.when(i == k)
            def _(k=k):
                rank_ref[k:k + 1, :] = rank_tok
        carry_ref[...] += jnp.sum(onehot_t, axis=1, keepdims=True)

    @pl.when(i == 3)
    def _():
        cnt2_ref[...] = carry_ref[...]
        padded = jnp.ceil(carry_ref[...] / blk) * blk
        rr = jax.lax.broadcasted_iota(jnp.int32, (nexp, nexp), 0)
        cc = jax.lax.broadcasted_iota(jnp.int32, (nexp, nexp), 1)
        ltri = (rr > cc).astype(jnp.float32)
        off_ref[...] = jnp.dot(ltri, padded, preferred_element_type=jnp.float32)

    @pl.when(i >= 4)
    def _():
        offv = off_ref[...]
        off_tok = jnp.sum(onehot_t * offv, axis=0, keepdims=True)
        any_e = jnp.sum(onehot_t, axis=0, keepdims=True) > 0.5
        for k in range(4):
            @pl.when(i == k + 4)
            def _(k=k):
                slot = off_tok + rank_ref[k:k + 1, :]
                slot_ref[0, :, :] = jnp.where(
                    any_e, slot.astype(jnp.int32), nslot - 1)


def _gffn_body(te_ref, oe_ref, x_ref, w1_ref, b1_ref, w2_ref, b2_ref, y_ref,
               *, blk):
    t = pl.program_id(0)
    x = x_ref[...]
    h = jnp.dot(x, w1_ref[0], preferred_element_type=jnp.float32)
    h = jnp.maximum(h + b1_ref[0], 0.0)
    y = jnp.dot(h, w2_ref[0], preferred_element_type=jnp.float32)
    y = y + b2_ref[0]
    row = jax.lax.broadcasted_iota(jnp.int32, y.shape, 0) + t * blk
    y_ref[...] = jnp.where(row < oe_ref[t], y, 0.0)


def kernel(x, Wg, bg, W1, b1, W2, b2):
    b, s, d = x.shape
    nexp = Wg.shape[1]
    total = b * s
    cap = int(total / nexp * 1.25)
    nt = 8
    tb = total // nt
    xf = x.reshape(total, d)

    idx8, psum = pl.pallas_call(
        _gate_body,
        grid=(nt,),
        in_specs=[
            pl.BlockSpec((tb, d), lambda i: (i, 0)),
            pl.BlockSpec((d, nexp), lambda i: (0, 0)),
            pl.BlockSpec((1, nexp), lambda i: (0, 0)),
        ],
        out_specs=[
            pl.BlockSpec((1, 1, tb), lambda i: (i, 0, 0)),
            pl.BlockSpec((1, nexp), lambda i: (0, 0)),
        ],
        out_shape=[
            jax.ShapeDtypeStruct((nt, 1, tb), jnp.int32),
            jax.ShapeDtypeStruct((1, nexp), jnp.float32),
        ],
    )(xf, Wg, bg.reshape(1, nexp))

    route = functools.partial(_route_body, nexp=nexp, tb=tb, cap=cap)
    tm8, cb0, ctot = pl.pallas_call(
        route,
        grid=(nt,),
        in_specs=[pl.BlockSpec((1, 1, tb), lambda i: (i, 0, 0))],
        out_specs=[
            pl.BlockSpec((1, 1, tb), lambda i: (i, 0, 0)),
            pl.BlockSpec((nexp, 1), lambda i: (0, 0)),
            pl.BlockSpec((nexp, 1), lambda i: (0, 0)),
        ],
        out_shape=[
            jax.ShapeDtypeStruct((nt, 1, tb), jnp.int32),
            jax.ShapeDtypeStruct((nexp, 1), jnp.float32),
            jax.ShapeDtypeStruct((nexp, 1), jnp.float32),
        ],
        scratch_shapes=[
            pltpu.VMEM((nexp, 1), jnp.float32),
            pltpu.VMEM((4, tb), jnp.int32),
        ],
    )(idx8)

    cb0f = cb0.reshape(nexp)
    ctotf = ctot.reshape(nexp)
    earr = jnp.arange(nexp)
    a0 = jnp.max(jnp.where(cb0f > 0.5, earr, -1))
    a1 = jnp.max(jnp.where((ctotf - cb0f > 0.5) & (cb0f < cap), earr, -1))
    tm = tm8.reshape(nt, tb)[nt // 2:].reshape(s)
    tm = tm.at[0].set(jnp.maximum(tm[0], a0))
    tm = tm.at[1].set(jnp.maximum(tm[1], a1))

    blk = 128
    nslot = s + nexp * blk
    ntile = nslot // blk
    nt2 = 4
    tb2 = s // nt2
    tm4 = tm.reshape(nt2, 1, tb2)

    group = functools.partial(
        _group_body, nexp=nexp, tb2=tb2, blk=blk, nslot=nslot)
    slot4, cnt2, off = pl.pallas_call(
        group,
        grid=(2 * nt2,),
        in_specs=[pl.BlockSpec((1, 1, tb2), lambda i: (i % nt2, 0, 0))],
        out_specs=[
            pl.BlockSpec((1, 1, tb2),
                         lambda i: (jnp.maximum(i - nt2, 0), 0, 0)),
            pl.BlockSpec((nexp, 1), lambda i: (0, 0)),
            pl.BlockSpec((nexp, 1), lambda i: (0, 0)),
        ],
        out_shape=[
            jax.ShapeDtypeStruct((nt2, 1, tb2), jnp.int32),
            jax.ShapeDtypeStruct((nexp, 1), jnp.float32),
            jax.ShapeDtypeStruct((nexp, 1), jnp.float32),
        ],
        scratch_shapes=[
            pltpu.VMEM((nexp, 1), jnp.float32),
            pltpu.VMEM((nt2, tb2), jnp.float32),
        ],
    )(tm4)

    slot = slot4.reshape(s)
    cnt2f = cnt2.reshape(nexp)
    offf = off.reshape(nexp)
    off_tiles = (offf / blk).astype(jnp.int32)
    t_iota = jnp.arange(ntile)
    te = jnp.sum(t_iota[:, None] >= off_tiles[None, :], axis=1) - 1
    te = te.astype(jnp.int32)
    oe = (offf + cnt2f).astype(jnp.int32)[te]

    xs = jnp.zeros((nslot, d), x.dtype).at[slot].set(x[0])

    gffn = functools.partial(_gffn_body, blk=blk)
    ys = pl.pallas_call(
        gffn,
        grid_spec=pltpu.PrefetchScalarGridSpec(
            num_scalar_prefetch=2,
            grid=(ntile,),
            in_specs=[
                pl.BlockSpec((blk, d), lambda t, te_r, oe_r: (t, 0)),
                pl.BlockSpec((1, d, d),
                             lambda t, te_r, oe_r: (te_r[t], 0, 0)),
                pl.BlockSpec((1, 1, d),
                             lambda t, te_r, oe_r: (te_r[t], 0, 0)),
                pl.BlockSpec((1, d, d),
                             lambda t, te_r, oe_r: (te_r[t], 0, 0)),
                pl.BlockSpec((1, 1, d),
                             lambda t, te_r, oe_r: (te_r[t], 0, 0)),
            ],
            out_specs=pl.BlockSpec((blk, d), lambda t, te_r, oe_r: (t, 0)),
        ),
        out_shape=jax.ShapeDtypeStruct((nslot, d), jnp.float32),
    )(te, oe, xs, W1, b1.reshape(nexp, 1, d), W2, b2.reshape(nexp, 1, d))

    y0 = jnp.take(ys, slot, axis=0)
    out = jnp.concatenate([y0[None], jnp.zeros_like(y0)[None]], axis=0)
    probs_mean = psum.reshape(nexp) / total
    density = ctotf / total
    expert_loss = jnp.mean(probs_mean * density * (nexp ** 2))
    return out, expert_loss


# speedup vs baseline: 6.4040x; 1.0753x over previous
import functools

import jax
import jax.numpy as jnp
from jax import lax
from jax.experimental import pallas as pl
from jax.experimental.pallas import tpu as pltpu
from jax.experimental.pallas import tpu_sc as plsc


def _gate_body(x_ref, wg_ref, bg_ref, idx_ref, psum_ref):
    i = pl.program_id(0)
    x = x_ref[...]
    logits = jnp.dot(x, wg_ref[...], preferred_element_type=jnp.float32)
    logits = logits + bg_ref[...]
    m = jnp.max(logits, axis=-1, keepdims=True)
    p = jnp.exp(logits - m)
    probs = p / jnp.sum(p, axis=-1, keepdims=True)
    lane = jax.lax.broadcasted_iota(jnp.int32, logits.shape, 1)
    nexp = logits.shape[-1]
    idx = jnp.min(jnp.where(logits == m, lane, nexp), axis=-1)
    idx_ref[0, 0, :] = idx

    @pl.when(i == 0)
    def _():
        psum_ref[...] = jnp.zeros_like(psum_ref)

    psum_ref[...] += jnp.sum(probs, axis=0, keepdims=True)


def _route_body(idx_ref, tm_ref, cb0_ref, ctot_ref, carry_ref, ke_ref, *,
                nexp, tb, cap):
    i = pl.program_id(0)

    @pl.when(i == 0)
    def _():
        carry_ref[...] = jnp.zeros_like(carry_ref)

    idxv = idx_ref[0, :, :]
    eio = jax.lax.broadcasted_iota(jnp.int32, (nexp, tb), 0)
    onehot_t = (idxv == eio).astype(jnp.float32)
    r = jax.lax.broadcasted_iota(jnp.int32, (tb, tb), 0)
    c = jax.lax.broadcasted_iota(jnp.int32, (tb, tb), 1)
    upper = (r < c).astype(jnp.float32)
    rank = jnp.dot(onehot_t, upper, preferred_element_type=jnp.float32)
    rank = rank + carry_ref[...]
    kept = (onehot_t > 0.5) & (rank < float(cap))
    ke = jnp.max(jnp.where(kept, eio, -1), axis=0, keepdims=True)
    carry_ref[...] += jnp.sum(onehot_t, axis=1, keepdims=True)

    @pl.when(i < 4)
    def _():
        tm_ref[0, :, :] = ke

    for k in range(4):
        @pl.when(i == k)
        def _(k=k):
            ke_ref[k:k + 1, :] = ke

        @pl.when(i == k + 4)
        def _(k=k):
            tm_ref[0, :, :] = jnp.maximum(ke_ref[k:k + 1, :], ke)

    @pl.when(i == 3)
    def _():
        cb0_ref[...] = carry_ref[...]

    @pl.when(i == 7)
    def _():
        ctot_ref[...] = carry_ref[...]


def _group_body(tm_ref, slot_ref, cnt2_ref, off_ref, carry_ref, rank_ref, *,
                nexp, tb2, blk, nslot):
    i = pl.program_id(0)

    @pl.when(i == 0)
    def _():
        carry_ref[...] = jnp.zeros_like(carry_ref)

    tmv = tm_ref[0, :, :]
    eio = jax.lax.broadcasted_iota(jnp.int32, (nexp, tb2), 0)
    onehot_t = (tmv == eio).astype(jnp.float32)

    @pl.when(i < 4)
    def _():
        r = jax.lax.broadcasted_iota(jnp.int32, (tb2, tb2), 0)
        c = jax.lax.broadcasted_iota(jnp.int32, (tb2, tb2), 1)
        upper = (r < c).astype(jnp.float32)
        rank = jnp.dot(onehot_t, upper, preferred_element_type=jnp.float32)
        rank = rank + carry_ref[...]
        rank_tok = jnp.sum(onehot_t * rank, axis=0, keepdims=True)
        for k in range(4):
            @pl.when(i == k)
            def _(k=k):
                rank_ref[k:k + 1, :] = rank_tok
        carry_ref[...] += jnp.sum(onehot_t, axis=1, keepdims=True)

    @pl.when(i == 3)
    def _():
        cnt2_ref[...] = carry_ref[...]
        padded = jnp.ceil(carry_ref[...] / blk) * blk
        rr = jax.lax.broadcasted_iota(jnp.int32, (nexp, nexp), 0)
        cc = jax.lax.broadcasted_iota(jnp.int32, (nexp, nexp), 1)
        ltri = (rr > cc).astype(jnp.float32)
        off_ref[...] = jnp.dot(ltri, padded, preferred_element_type=jnp.float32)

    @pl.when(i >= 4)
    def _():
        offv = off_ref[...]
        off_tok = jnp.sum(onehot_t * offv, axis=0, keepdims=True)
        any_e = jnp.sum(onehot_t, axis=0, keepdims=True) > 0.5
        for k in range(4):
            @pl.when(i == k + 4)
            def _(k=k):
                slot = off_tok + rank_ref[k:k + 1, :]
                slot_ref[0, :, :] = jnp.where(
                    any_e, slot.astype(jnp.int32), nslot - 1)


def _dispatch_body(x_hbm, slot_hbm, xs_hbm, idx_v, rows_v, sem, *, nc, rw):
    wid = lax.axis_index("s") * nc + lax.axis_index("c")
    pltpu.sync_copy(slot_hbm.at[wid], idx_v)
    pltpu.sync_copy(x_hbm.at[pl.ds(wid * rw, rw)], rows_v)
    pltpu.async_copy(rows_v, xs_hbm.at[idx_v], sem).wait()


def _combine_body(ys_hbm, slot_hbm, y0_hbm, idx_v, rows_v, sem, *, nc, rw):
    wid = lax.axis_index("s") * nc + lax.axis_index("c")
    pltpu.sync_copy(slot_hbm.at[wid], idx_v)
    pltpu.async_copy(ys_hbm.at[idx_v], rows_v, sem).wait()
    pltpu.sync_copy(rows_v, y0_hbm.at[pl.ds(wid * rw, rw)])


def _gffn_body(te_ref, oe_ref, x_ref, w1_ref, b1_ref, w2_ref, b2_ref, y_ref,
               *, blk):
    t = pl.program_id(0)
    x = x_ref[...]
    h = jnp.dot(x, w1_ref[0], preferred_element_type=jnp.float32)
    h = jnp.maximum(h + b1_ref[0], 0.0)
    y = jnp.dot(h, w2_ref[0], preferred_element_type=jnp.float32)
    y = y + b2_ref[0]
    row = jax.lax.broadcasted_iota(jnp.int32, y.shape, 0) + t * blk
    y_ref[...] = jnp.where(row < oe_ref[t], y, 0.0)


def kernel(x, Wg, bg, W1, b1, W2, b2):
    b, s, d = x.shape
    nexp = Wg.shape[1]
    total = b * s
    cap = int(total / nexp * 1.25)
    nt = 8
    tb = total // nt
    xf = x.reshape(total, d)

    idx8, psum = pl.pallas_call(
        _gate_body,
        grid=(nt,),
        in_specs=[
            pl.BlockSpec((tb, d), lambda i: (i, 0)),
            pl.BlockSpec((d, nexp), lambda i: (0, 0)),
            pl.BlockSpec((1, nexp), lambda i: (0, 0)),
        ],
        out_specs=[
            pl.BlockSpec((1, 1, tb), lambda i: (i, 0, 0)),
            pl.BlockSpec((1, nexp), lambda i: (0, 0)),
        ],
        out_shape=[
            jax.ShapeDtypeStruct((nt, 1, tb), jnp.int32),
            jax.ShapeDtypeStruct((1, nexp), jnp.float32),
        ],
    )(xf, Wg, bg.reshape(1, nexp))

    route = functools.partial(_route_body, nexp=nexp, tb=tb, cap=cap)
    tm8, cb0, ctot = pl.pallas_call(
        route,
        grid=(nt,),
        in_specs=[pl.BlockSpec((1, 1, tb), lambda i: (i, 0, 0))],
        out_specs=[
            pl.BlockSpec((1, 1, tb), lambda i: (i, 0, 0)),
            pl.BlockSpec((nexp, 1), lambda i: (0, 0)),
            pl.BlockSpec((nexp, 1), lambda i: (0, 0)),
        ],
        out_shape=[
            jax.ShapeDtypeStruct((nt, 1, tb), jnp.int32),
            jax.ShapeDtypeStruct((nexp, 1), jnp.float32),
            jax.ShapeDtypeStruct((nexp, 1), jnp.float32),
        ],
        scratch_shapes=[
            pltpu.VMEM((nexp, 1), jnp.float32),
            pltpu.VMEM((4, tb), jnp.int32),
        ],
    )(idx8)

    cb0f = cb0.reshape(nexp)
    ctotf = ctot.reshape(nexp)
    earr = jnp.arange(nexp)
    a0 = jnp.max(jnp.where(cb0f > 0.5, earr, -1))
    a1 = jnp.max(jnp.where((ctotf - cb0f > 0.5) & (cb0f < cap), earr, -1))
    tm = tm8.reshape(nt, tb)[nt // 2:].reshape(s)
    tm = tm.at[0].set(jnp.maximum(tm[0], a0))
    tm = tm.at[1].set(jnp.maximum(tm[1], a1))

    blk = 128
    nslot = s + nexp * blk
    ntile = nslot // blk
    nt2 = 4
    tb2 = s // nt2
    tm4 = tm.reshape(nt2, 1, tb2)

    group = functools.partial(
        _group_body, nexp=nexp, tb2=tb2, blk=blk, nslot=nslot)
    slot4, cnt2, off = pl.pallas_call(
        group,
        grid=(2 * nt2,),
        in_specs=[pl.BlockSpec((1, 1, tb2), lambda i: (i % nt2, 0, 0))],
        out_specs=[
            pl.BlockSpec((1, 1, tb2),
                         lambda i: (jnp.maximum(i - nt2, 0), 0, 0)),
            pl.BlockSpec((nexp, 1), lambda i: (0, 0)),
            pl.BlockSpec((nexp, 1), lambda i: (0, 0)),
        ],
        out_shape=[
            jax.ShapeDtypeStruct((nt2, 1, tb2), jnp.int32),
            jax.ShapeDtypeStruct((nexp, 1), jnp.float32),
            jax.ShapeDtypeStruct((nexp, 1), jnp.float32),
        ],
        scratch_shapes=[
            pltpu.VMEM((nexp, 1), jnp.float32),
            pltpu.VMEM((nt2, tb2), jnp.float32),
        ],
    )(tm4)

    slot = slot4.reshape(s)
    cnt2f = cnt2.reshape(nexp)
    offf = off.reshape(nexp)
    off_tiles = (offf / blk).astype(jnp.int32)
    t_iota = jnp.arange(ntile)
    te = jnp.sum(t_iota[:, None] >= off_tiles[None, :], axis=1) - 1
    te = te.astype(jnp.int32)
    oe = (offf + cnt2f).astype(jnp.int32)[te]

    info = plsc.get_sparse_core_info()
    nc, ns = info.num_cores, info.num_subcores
    nw = nc * ns
    rw = s // nw
    slot_w = slot.reshape(nw, rw)
    mesh = plsc.VectorSubcoreMesh(core_axis_name="c", subcore_axis_name="s")

    xs = pl.kernel(
        functools.partial(_dispatch_body, nc=nc, rw=rw),
        out_type=jax.ShapeDtypeStruct((nslot, d), jnp.float32),
        mesh=mesh,
        scratch_types=[
            pltpu.VMEM((rw,), jnp.int32),
            pltpu.VMEM((rw, d), jnp.float32),
            pltpu.SemaphoreType.DMA,
        ],
    )(x[0], slot_w)

    gffn = functools.partial(_gffn_body, blk=blk)
    ys = pl.pallas_call(
        gffn,
        grid_spec=pltpu.PrefetchScalarGridSpec(
            num_scalar_prefetch=2,
            grid=(ntile,),
            in_specs=[
                pl.BlockSpec((blk, d), lambda t, te_r, oe_r: (t, 0)),
                pl.BlockSpec((1, d, d),
                             lambda t, te_r, oe_r: (te_r[t], 0, 0)),
                pl.BlockSpec((1, 1, d),
                             lambda t, te_r, oe_r: (te_r[t], 0, 0)),
                pl.BlockSpec((1, d, d),
                             lambda t, te_r, oe_r: (te_r[t], 0, 0)),
                pl.BlockSpec((1, 1, d),
                             lambda t, te_r, oe_r: (te_r[t], 0, 0)),
            ],
            out_specs=pl.BlockSpec((blk, d), lambda t, te_r, oe_r: (t, 0)),
        ),
        out_shape=jax.ShapeDtypeStruct((nslot, d), jnp.float32),
    )(te, oe, xs, W1, b1.reshape(nexp, 1, d), W2, b2.reshape(nexp, 1, d))

    y0 = pl.kernel(
        functools.partial(_combine_body, nc=nc, rw=rw),
        out_type=jax.ShapeDtypeStruct((s, d), jnp.float32),
        mesh=mesh,
        scratch_types=[
            pltpu.VMEM((rw,), jnp.int32),
            pltpu.VMEM((rw, d), jnp.float32),
            pltpu.SemaphoreType.DMA,
        ],
    )(ys, slot_w)
    out = jnp.concatenate([y0[None], jnp.zeros_like(y0)[None]], axis=0)
    probs_mean = psum.reshape(nexp) / total
    density = ctotf / total
    expert_loss = jnp.mean(probs_mean * density * (nexp ** 2))
    return out, expert_loss
